# baseline (speedup 1.0000x reference)
import functools

import jax
import jax.numpy as jnp
from jax import lax
from jax.experimental import pallas as pl
from jax.experimental.pallas import tpu as pltpu
from jax.experimental.pallas import tpu_sc as plsc

_N = 2048
_C = 2048
_H = 16
_D = 128
_U = 40
_SCALE = _D ** -0.5


def _q_kernel(x_ref, w_ref, b_ref, q_ref, idx_ref, idx0_ref, xbf_ref, n2_ref):
    hh = pl.program_id(0)

    @pl.when(hh == 0)
    def _():
        xbf_ref[...] = x_ref[...].astype(jnp.bfloat16)

    acc = jax.lax.dot_general(
        xbf_ref[...], w_ref[...].astype(jnp.bfloat16), (((1,), (0,)), ((), ())),
        preferred_element_type=jnp.float32)
    acc = acc + b_ref[0]
    for j in range(4):
        q_ref[j, :, :] = acc[:, j * _D:(j + 1) * _D]
    sq = acc * acc
    cj = jax.lax.broadcasted_iota(jnp.int32, (4, 512), 0)
    cc = jax.lax.broadcasted_iota(jnp.int32, (4, 512), 1)
    sel = (cc // _D == cj).astype(jnp.float32)
    n2_ref[hh] = jax.lax.dot_general(
        sel, sq, (((1,), (1,)), ((), ())), preferred_element_type=jnp.float32)

    @pl.when(hh == 3)
    def _():
        bits = jax.lax.bitcast_convert_type(n2_ref[...], jnp.int32)
        col = jax.lax.broadcasted_iota(jnp.int32, (4, 4, _N), 2)
        keys = (bits & ~jnp.int32(2047)) | (jnp.int32(2047) - col)
        picks = []
        for _ in range(_U):
            m = jnp.max(keys, axis=2, keepdims=True)
            picks.append(m)
            keys = jnp.where(keys == m, jnp.iinfo(jnp.int32).min, keys)
        mkeys = jnp.concatenate(picks, axis=2)
        idx = jnp.int32(2047) - (mkeys & jnp.int32(2047))
        head = (jax.lax.broadcasted_iota(jnp.int32, (4, 4, _U), 0) * 4
                + jax.lax.broadcasted_iota(jnp.int32, (4, 4, _U), 1))
        idx_ref[...] = idx + head * _N
        one = jnp.ones((1, 1), jnp.float32)
        idx0_ref[...] = jax.lax.dot_general(
            idx[0, 0:1, :].astype(jnp.float32), one,
            (((0,), (0,)), ((), ())), preferred_element_type=jnp.float32)


def _q_proj(x, w, b):
    return pl.pallas_call(
        _q_kernel,
        grid=(4,),
        in_specs=[
            pl.BlockSpec((_N, _C), lambda hh: (0, 0)),
            pl.BlockSpec((_C, 512), lambda hh: (0, hh)),
            pl.BlockSpec((1, 1, 512), lambda hh: (hh, 0, 0)),
        ],
        out_specs=[
            pl.BlockSpec((4, _N, _D), lambda hh: (hh, 0, 0)),
            pl.BlockSpec((4, 4, _U), lambda hh: (0, 0, 0)),
            pl.BlockSpec((_U, 1), lambda hh: (0, 0)),
        ],
        out_shape=[
            jax.ShapeDtypeStruct((_H, _N, _D), jnp.float32),
            jax.ShapeDtypeStruct((4, 4, _U), jnp.int32),
            jax.ShapeDtypeStruct((_U, 1), jnp.float32),
        ],
        scratch_shapes=[
            pltpu.VMEM((_N, _C), jnp.bfloat16),
            pltpu.VMEM((4, 4, _N), jnp.float32),
        ],
    )(x, w, b)


def _kv_kernel(x_ref, w_ref, b_ref, k_ref, v_ref, xbf_ref):
    hh = pl.program_id(0)
    t = pl.program_id(1)

    @pl.when((hh == 0) & (t == 0))
    def _():
        xbf_ref[...] = x_ref[...].astype(jnp.bfloat16)

    acc = jax.lax.dot_general(
        xbf_ref[...], w_ref[...].astype(jnp.bfloat16), (((1,), (0,)), ((), ())),
        preferred_element_type=jnp.float32)
    acc = acc + b_ref[0]

    @pl.when(t == 0)
    def _():
        for j in range(4):
            k_ref[j, :, :] = acc[:, j * _D:(j + 1) * _D]

    @pl.when(t == 1)
    def _():
        for j in range(4):
            v_ref[j, :, :] = acc[:, j * _D:(j + 1) * _D].astype(jnp.bfloat16)


def _kv_proj(x, w, b):
    return pl.pallas_call(
        _kv_kernel,
        grid=(4, 2),
        in_specs=[
            pl.BlockSpec((_N, _C), lambda hh, t: (0, 0)),
            pl.BlockSpec((_C, 512), lambda hh, t: (0, (t + 1) * 4 + hh)),
            pl.BlockSpec((1, 1, 512), lambda hh, t: ((t + 1) * 4 + hh, 0, 0)),
        ],
        out_specs=[
            pl.BlockSpec((4, _N, _D), lambda hh, t: (hh, 0, 0)),
            pl.BlockSpec((4, _N, _D), lambda hh, t: (hh, 0, 0)),
        ],
        out_shape=[
            jax.ShapeDtypeStruct((_H, _N, _D), jnp.float32),
            jax.ShapeDtypeStruct((_H, _N, _D), jnp.bfloat16),
        ],
        scratch_shapes=[pltpu.VMEM((_N, _C), jnp.bfloat16)],
    )(x, w, b)


def _sc_gather(q_rows, idx_flat):
    mesh = plsc.VectorSubcoreMesh(core_axis_name="c", subcore_axis_name="s")

    @functools.partial(
        pl.kernel, mesh=mesh,
        out_type=jax.ShapeDtypeStruct((_H * _U, _D), jnp.float32),
        scratch_types=[
            pltpu.VMEM((_U,), jnp.int32),
            pltpu.VMEM((_U, _D), jnp.float32),
            pltpu.SemaphoreType.DMA,
        ],
    )
    def gather(table_hbm, idx_hbm, out_hbm, idx_v, rows_v, sem):
        wid = lax.axis_index("s") * 2 + lax.axis_index("c")

        @pl.when(wid < _H)
        def _():
            base = wid * _U
            pltpu.sync_copy(idx_hbm.at[pl.ds(base, _U)], idx_v)
            pltpu.async_copy(table_hbm.at[idx_v], rows_v, sem).wait()
            pltpu.sync_copy(rows_v, out_hbm.at[pl.ds(base, _U)])

    return gather(q_rows, idx_flat)


def _attn_kernel(qred_ref, k_ref, v_ref, idx_ref, idx0_ref, out_ref):
    h = pl.program_id(0)
    q_red = qred_ref[0]
    k = k_ref[0]
    v = v_ref[0]
    s = jax.lax.dot_general(
        q_red, k, (((1,), (1,)), ((), ())),
        preferred_element_type=jnp.float32) * _SCALE

    p = s - jnp.max(s, axis=1, keepdims=True)
    p = jnp.exp(p)
    p = p / jnp.sum(p, axis=1, keepdims=True)
    p2 = p - jnp.max(p, axis=1, keepdims=True)
    p2 = jnp.exp(p2)
    p2 = p2 / jnp.sum(p2, axis=1, keepdims=True)

    rows = jax.lax.dot_general(
        p2.astype(jnp.bfloat16), v, (((1,), (0,)), ((), ())),
        preferred_element_type=jnp.float32)
    rows_aug = jnp.concatenate(
        [rows, jnp.ones((_U, _D), jnp.float32)], axis=1)

    local_idx = (idx_ref[0, 0:1, :] - h * _N).astype(jnp.float32)
    match = (idx0_ref[...] == local_idx).astype(jnp.float32)
    out_ref[0] = jax.lax.dot_general(
        match, rows_aug, (((1,), (0,)), ((), ())),
        preferred_element_type=jnp.float32)


def _sparse_attn(q_red, k, v, idx_flat, idx0_col):
    return pl.pallas_call(
        _attn_kernel,
        grid=(_H,),
        in_specs=[
            pl.BlockSpec((1, _U, _D), lambda h: (h, 0, 0)),
            pl.BlockSpec((1, _N, _D), lambda h: (h, 0, 0)),
            pl.BlockSpec((1, _N, _D), lambda h: (h, 0, 0)),
            pl.BlockSpec((1, 1, _U), lambda h: (h, 0, 0)),
            pl.BlockSpec((_U, 1), lambda h: (0, 0)),
        ],
        out_specs=pl.BlockSpec((1, _U, 2 * _D), lambda h: (h, 0, 0)),
        out_shape=jax.ShapeDtypeStruct((_H, _U, 2 * _D), jnp.float32),
    )(q_red, k, v, idx_flat, idx0_col)


def _fc_kernel(cand_ref, idx_ref, w_ref, b_ref, out_ref, aug_ref):
    i = pl.program_id(0)

    @pl.when(i == 0)
    def _():
        cand = jnp.concatenate(
            [cand_ref[h, :, :_D] for h in range(_H)], axis=1)
        inter = cand_ref[0, :, _D:_D + 1]
        for h in range(1, _H):
            inter = jnp.minimum(inter, cand_ref[h, :, _D:_D + 1])
        fc = jax.lax.dot_general(
            cand, w_ref[...], (((1,), (0,)), ((), ())),
            preferred_element_type=jnp.float32) + b_ref[...]
        aug_ref[...] = jnp.concatenate(
            [fc, inter * jnp.ones((_U, _D), jnp.float32)], axis=1)

    idx0 = idx_ref[0, 0:1, :].astype(jnp.float32)
    row = (jax.lax.broadcasted_iota(jnp.int32, (512, _U), 0)
           + i * 512).astype(jnp.float32)
    onehot = (row == idx0).astype(jnp.float32)
    scat = jax.lax.dot_general(
        onehot, aug_ref[...], (((1,), (0,)), ((), ())),
        preferred_element_type=jnp.float32)
    out_ref[...] = jnp.where(scat[:, _C:_C + 1] > 0.5, scat[:, :_C], jnp.nan)


def _fc(cand, idx_flat, w, b):
    return pl.pallas_call(
        _fc_kernel,
        grid=(4,),
        in_specs=[
            pl.BlockSpec((_H, _U, 2 * _D), lambda i: (0, 0, 0)),
            pl.BlockSpec((1, 1, _U), lambda i: (0, 0, 0)),
            pl.BlockSpec((_C, _C), lambda i: (0, 0)),
            pl.BlockSpec((1, _C), lambda i: (0, 0)),
        ],
        out_specs=pl.BlockSpec((512, _C), lambda i: (i, 0)),
        out_shape=jax.ShapeDtypeStruct((_N, _C), jnp.float32),
        scratch_shapes=[pltpu.VMEM((_U, _C + _D), jnp.float32)],
    )(cand, idx_flat, w, b)


@jax.jit
def kernel(query, W_qkv, b_qkv, W_fc, b_fc):
    B, N, C = query.shape
    b3 = b_qkv.reshape(12, 1, 512)
    x = query.reshape(N, C)
    q, idx3, idx0_col = _q_proj(x, W_qkv, b3)
    idx_flat = idx3.reshape(_H, 1, _U)
    q_red = _sc_gather(q.reshape(_H * _N, _D), idx3.reshape(_H * _U))
    k, v = _kv_proj(x, W_qkv, b3)
    cand = _sparse_attn(q_red.reshape(_H, _U, _D), k, v, idx_flat, idx0_col)
    out = _fc(cand, idx_flat, W_fc, b_fc.reshape(1, C))
    return out.reshape(B, N, C)


# speedup vs baseline: 1.0391x; 1.0391x over previous
import functools

import jax
import jax.numpy as jnp
from jax import lax
from jax.experimental import pallas as pl
from jax.experimental.pallas import tpu as pltpu
from jax.experimental.pallas import tpu_sc as plsc

_N = 2048
_C = 2048
_H = 16
_D = 128
_U = 40
_SCALE = _D ** -0.5


def _qkv_kernel(x_ref, w_ref, b_ref, q_ref, k_ref, v_ref, idx_ref, idx0_ref,
                xbf_ref, n2_ref):
    hh = pl.program_id(0)
    s = pl.program_id(1)

    @pl.when((hh == 0) & (s == 0))
    def _():
        xbf_ref[...] = x_ref[...].astype(jnp.bfloat16)

    acc = jax.lax.dot_general(
        xbf_ref[...], w_ref[...].astype(jnp.bfloat16), (((1,), (0,)), ((), ())),
        preferred_element_type=jnp.float32)
    acc = acc + b_ref[0]

    @pl.when(s == 0)
    def _():
        for j in range(4):
            q_ref[j, :, :] = acc[:, j * _D:(j + 1) * _D]
        sq = acc * acc
        cj = jax.lax.broadcasted_iota(jnp.int32, (4, 512), 0)
        cc = jax.lax.broadcasted_iota(jnp.int32, (4, 512), 1)
        sel = (cc // _D == cj).astype(jnp.float32)
        n2_ref[hh] = jax.lax.dot_general(
            sel, sq, (((1,), (1,)), ((), ())), preferred_element_type=jnp.float32)

    @pl.when(s == 1)
    def _():
        for j in range(4):
            k_ref[j, :, :] = acc[:, j * _D:(j + 1) * _D]

    @pl.when(s == 2)
    def _():
        for j in range(4):
            v_ref[j, :, :] = acc[:, j * _D:(j + 1) * _D].astype(jnp.bfloat16)

    @pl.when((hh == 3) & (s == 2))
    def _():
        bits = jax.lax.bitcast_convert_type(n2_ref[...], jnp.int32)
        col = jax.lax.broadcasted_iota(jnp.int32, (4, 4, _N), 2)
        keys = (bits & ~jnp.int32(2047)) | (jnp.int32(2047) - col)
        picks = []
        for _ in range(_U):
            m = jnp.max(keys, axis=2, keepdims=True)
            picks.append(m)
            keys = jnp.where(keys == m, jnp.iinfo(jnp.int32).min, keys)
        mkeys = jnp.concatenate(picks, axis=2)
        idx = jnp.int32(2047) - (mkeys & jnp.int32(2047))
        head = (jax.lax.broadcasted_iota(jnp.int32, (4, 4, _U), 0) * 4
                + jax.lax.broadcasted_iota(jnp.int32, (4, 4, _U), 1))
        idx_ref[...] = idx + head * _N
        one = jnp.ones((1, 1), jnp.float32)
        idx0_ref[...] = jax.lax.dot_general(
            idx[0, 0:1, :].astype(jnp.float32), one,
            (((0,), (0,)), ((), ())), preferred_element_type=jnp.float32)


def _qkv_proj(x, w, b):
    return pl.pallas_call(
        _qkv_kernel,
        grid=(4, 3),
        in_specs=[
            pl.BlockSpec((_N, _C), lambda hh, s: (0, 0)),
            pl.BlockSpec((_C, 512), lambda hh, s: (0, s * 4 + hh)),
            pl.BlockSpec((1, 1, 512), lambda hh, s: (s * 4 + hh, 0, 0)),
        ],
        out_specs=[
            pl.BlockSpec((4, _N, _D), lambda hh, s: (hh, 0, 0)),
            pl.BlockSpec((4, _N, _D), lambda hh, s: (hh, 0, 0)),
            pl.BlockSpec((4, _N, _D), lambda hh, s: (hh, 0, 0)),
            pl.BlockSpec((4, 4, _U), lambda hh, s: (0, 0, 0)),
            pl.BlockSpec((_U, 1), lambda hh, s: (0, 0)),
        ],
        out_shape=[
            jax.ShapeDtypeStruct((_H, _N, _D), jnp.float32),
            jax.ShapeDtypeStruct((_H, _N, _D), jnp.float32),
            jax.ShapeDtypeStruct((_H, _N, _D), jnp.bfloat16),
            jax.ShapeDtypeStruct((4, 4, _U), jnp.int32),
            jax.ShapeDtypeStruct((_U, 1), jnp.float32),
        ],
        scratch_shapes=[
            pltpu.VMEM((_N, _C), jnp.bfloat16),
            pltpu.VMEM((4, 4, _N), jnp.float32),
        ],
    )(x, w, b)


def _sc_gather(q_rows, idx_flat):
    mesh = plsc.VectorSubcoreMesh(core_axis_name="c", subcore_axis_name="s")

    @functools.partial(
        pl.kernel, mesh=mesh,
        out_type=jax.ShapeDtypeStruct((_H * _U, _D), jnp.float32),
        scratch_types=[
            pltpu.VMEM((_U,), jnp.int32),
            pltpu.VMEM((_U, _D), jnp.float32),
            pltpu.SemaphoreType.DMA,
        ],
    )
    def gather(table_hbm, idx_hbm, out_hbm, idx_v, rows_v, sem):
        wid = lax.axis_index("s") * 2 + lax.axis_index("c")

        @pl.when(wid < _H)
        def _():
            base = wid * _U
            pltpu.sync_copy(idx_hbm.at[pl.ds(base, _U)], idx_v)
            pltpu.async_copy(table_hbm.at[idx_v], rows_v, sem).wait()
            pltpu.sync_copy(rows_v, out_hbm.at[pl.ds(base, _U)])

    return gather(q_rows, idx_flat)


def _attn_fc_kernel(qred_ref, k_ref, v_ref, idx_ref, idx0h_ref, idx0_ref,
                    w_ref, b_ref, out_ref, aug_ref):
    h = pl.program_id(0)
    q_red = qred_ref[0]
    k = k_ref[0]
    v = v_ref[0]
    s = jax.lax.dot_general(
        q_red, k, (((1,), (1,)), ((), ())),
        preferred_element_type=jnp.float32) * _SCALE

    p = s - jnp.max(s, axis=1, keepdims=True)
    p = jnp.exp(p)
    p = p / jnp.sum(p, axis=1, keepdims=True)
    p2 = p - jnp.max(p, axis=1, keepdims=True)
    p2 = jnp.exp(p2)
    p2 = p2 / jnp.sum(p2, axis=1, keepdims=True)

    rows = jax.lax.dot_general(
        p2.astype(jnp.bfloat16), v, (((1,), (0,)), ((), ())),
        preferred_element_type=jnp.float32)
    rows_aug = jnp.concatenate(
        [rows, jnp.ones((_U, _D), jnp.float32)], axis=1)

    local_idx = (idx_ref[0, 0:1, :] - h * _N).astype(jnp.float32)
    match = (idx0_ref[...] == local_idx).astype(jnp.float32)
    aug_ref[h] = jax.lax.dot_general(
        match, rows_aug, (((1,), (0,)), ((), ())),
        preferred_element_type=jnp.float32)

    @pl.when(h == _H - 1)
    def _():
        cand = jnp.concatenate(
            [aug_ref[g, :, :_D] for g in range(_H)], axis=1)
        inter = aug_ref[0, :, _D:_D + 1]
        for g in range(1, _H):
            inter = jnp.minimum(inter, aug_ref[g, :, _D:_D + 1])
        fc = jax.lax.dot_general(
            cand, w_ref[...], (((1,), (0,)), ((), ())),
            preferred_element_type=jnp.float32) + b_ref[...]
        aug2 = jnp.concatenate(
            [fc, inter * jnp.ones((_U, _D), jnp.float32)], axis=1)
        idx0 = idx0h_ref[0, 0:1, :].astype(jnp.float32)
        row = jax.lax.broadcasted_iota(
            jnp.int32, (_N, _U), 0).astype(jnp.float32)
        onehot = (row == idx0).astype(jnp.float32)
        scat = jax.lax.dot_general(
            onehot, aug2, (((1,), (0,)), ((), ())),
            preferred_element_type=jnp.float32)
        out_ref[...] = jnp.where(
            scat[:, _C:_C + 1] > 0.5, scat[:, :_C], jnp.nan)


def _attn_fc(q_red, k, v, idx_flat, idx0_col, w, b):
    return pl.pallas_call(
        _attn_fc_kernel,
        grid=(_H,),
        in_specs=[
            pl.BlockSpec((1, _U, _D), lambda h: (h, 0, 0)),
            pl.BlockSpec((1, _N, _D), lambda h: (h, 0, 0)),
            pl.BlockSpec((1, _N, _D), lambda h: (h, 0, 0)),
            pl.BlockSpec((1, 1, _U), lambda h: (h, 0, 0)),
            pl.BlockSpec((1, 1, _U), lambda h: (0, 0, 0)),
            pl.BlockSpec((_U, 1), lambda h: (0, 0)),
            pl.BlockSpec((_C, _C), lambda h: (0, 0)),
            pl.BlockSpec((1, _C), lambda h: (0, 0)),
        ],
        out_specs=pl.BlockSpec((_N, _C), lambda h: (0, 0)),
        out_shape=jax.ShapeDtypeStruct((_N, _C), jnp.float32),
        scratch_shapes=[pltpu.VMEM((_H, _U, 2 * _D), jnp.float32)],
    )(q_red, k, v, idx_flat, idx_flat, idx0_col, w, b)


@jax.jit
def kernel(query, W_qkv, b_qkv, W_fc, b_fc):
    B, N, C = query.shape
    x = query.reshape(N, C)
    q, k, v, idx3, idx0_col = _qkv_proj(x, W_qkv, b_qkv.reshape(12, 1, 512))
    idx_flat = idx3.reshape(_H, 1, _U)
    q_red = _sc_gather(q.reshape(_H * _N, _D), idx3.reshape(_H * _U))
    out = _attn_fc(q_red.reshape(_H, _U, _D), k, v, idx_flat, idx0_col,
                   W_fc, b_fc.reshape(1, C))
    return out.reshape(B, N, C)


# speedup vs baseline: 1.0862x; 1.0453x over previous
import functools

import jax
import jax.numpy as jnp
from jax import lax
from jax.experimental import pallas as pl
from jax.experimental.pallas import tpu as pltpu
from jax.experimental.pallas import tpu_sc as plsc

_N = 2048
_C = 2048
_H = 16
_D = 128
_U = 40
_SCALE = _D ** -0.5


def _qkv_kernel(x_ref, w_ref, b_ref, q_ref, k_ref, v_ref, idx_ref, idx0_ref,
                xbf_ref, n2_ref):
    hh = pl.program_id(0)
    s = pl.program_id(1)

    @pl.when((hh == 0) & (s == 0))
    def _():
        xbf_ref[...] = x_ref[...].astype(jnp.bfloat16)

    acc = jax.lax.dot_general(
        xbf_ref[...], w_ref[...].astype(jnp.bfloat16), (((1,), (0,)), ((), ())),
        preferred_element_type=jnp.float32)
    acc = acc + b_ref[0]

    @pl.when(s == 0)
    def _():
        for j in range(4):
            q_ref[j, :, :] = acc[:, j * _D:(j + 1) * _D]
        sq = acc * acc
        cj = jax.lax.broadcasted_iota(jnp.int32, (4, 512), 0)
        cc = jax.lax.broadcasted_iota(jnp.int32, (4, 512), 1)
        sel = (cc // _D == cj).astype(jnp.float32)
        n2_ref[hh] = jax.lax.dot_general(
            sel, sq, (((1,), (1,)), ((), ())), preferred_element_type=jnp.float32)

    @pl.when(s == 1)
    def _():
        for j in range(4):
            k_ref[j, :, :] = acc[:, j * _D:(j + 1) * _D]

    @pl.when(s == 2)
    def _():
        for j in range(4):
            v_ref[j, :, :] = acc[:, j * _D:(j + 1) * _D].astype(jnp.bfloat16)

    @pl.when((hh == 3) & (s == 2))
    def _():
        bits = jax.lax.bitcast_convert_type(n2_ref[...], jnp.int32)
        col = jax.lax.broadcasted_iota(jnp.int32, (4, 4, _N), 2)
        keys = (bits & ~jnp.int32(2047)) | (jnp.int32(2047) - col)
        picks = []
        for _ in range(_U):
            m = jnp.max(keys, axis=2, keepdims=True)
            picks.append(m)
            keys = jnp.where(keys == m, jnp.iinfo(jnp.int32).min, keys)
        mkeys = jnp.concatenate(picks, axis=2)
        idx = jnp.int32(2047) - (mkeys & jnp.int32(2047))
        head = (jax.lax.broadcasted_iota(jnp.int32, (4, 4, _U), 0) * 4
                + jax.lax.broadcasted_iota(jnp.int32, (4, 4, _U), 1))
        idx_ref[...] = idx + head * _N
        one = jnp.ones((1, 1), jnp.float32)
        idx0_ref[...] = jax.lax.dot_general(
            idx[0, 0:1, :].astype(jnp.float32), one,
            (((0,), (0,)), ((), ())), preferred_element_type=jnp.float32)


def _qkv_proj(x, w, b):
    return pl.pallas_call(
        _qkv_kernel,
        grid=(4, 3),
        in_specs=[
            pl.BlockSpec((_N, _C), lambda hh, s: (0, 0)),
            pl.BlockSpec((_C, 512), lambda hh, s: (0, s * 4 + hh)),
            pl.BlockSpec((1, 1, 512), lambda hh, s: (s * 4 + hh, 0, 0)),
        ],
        out_specs=[
            pl.BlockSpec((4, _N, _D), lambda hh, s: (hh, 0, 0)),
            pl.BlockSpec((4, _N, _D), lambda hh, s: (hh, 0, 0)),
            pl.BlockSpec((4, _N, _D), lambda hh, s: (hh, 0, 0)),
            pl.BlockSpec((4, 4, _U), lambda hh, s: (0, 0, 0)),
            pl.BlockSpec((_U, 1), lambda hh, s: (0, 0)),
        ],
        out_shape=[
            jax.ShapeDtypeStruct((_H, _N, _D), jnp.float32),
            jax.ShapeDtypeStruct((_H, _N, _D), jnp.float32),
            jax.ShapeDtypeStruct((_H, _N, _D), jnp.bfloat16),
            jax.ShapeDtypeStruct((4, 4, _U), jnp.int32),
            jax.ShapeDtypeStruct((_U, 1), jnp.float32),
        ],
        scratch_shapes=[
            pltpu.VMEM((_N, _C), jnp.bfloat16),
            pltpu.VMEM((4, 4, _N), jnp.float32),
        ],
    )(x, w, b)


def _sc_gather(q_rows, idx_flat):
    mesh = plsc.VectorSubcoreMesh(core_axis_name="c", subcore_axis_name="s")

    @functools.partial(
        pl.kernel, mesh=mesh,
        out_type=jax.ShapeDtypeStruct((_H * _U, _D), jnp.float32),
        scratch_types=[
            pltpu.VMEM((_U,), jnp.int32),
            pltpu.VMEM((_U, _D), jnp.float32),
            pltpu.SemaphoreType.DMA,
        ],
    )
    def gather(table_hbm, idx_hbm, out_hbm, idx_v, rows_v, sem):
        wid = lax.axis_index("s") * 2 + lax.axis_index("c")

        @pl.when(wid < _H)
        def _():
            base = wid * _U
            pltpu.sync_copy(idx_hbm.at[pl.ds(base, _U)], idx_v)
            pltpu.async_copy(table_hbm.at[idx_v], rows_v, sem).wait()
            pltpu.sync_copy(rows_v, out_hbm.at[pl.ds(base, _U)])

    return gather(q_rows, idx_flat)


def _attn_fc_kernel(qred_ref, k_ref, v_ref, idx_ref, idx0h_ref, idx0_ref,
                    w_ref, b_ref, out_ref, aug_ref, wsc_ref, aug2_ref, sem):
    h = pl.program_id(0)
    q_red = qred_ref[0]
    k = k_ref[0]
    v = v_ref[0]
    s = jax.lax.dot_general(
        q_red, k, (((1,), (1,)), ((), ())),
        preferred_element_type=jnp.float32) * _SCALE

    p = s - jnp.max(s, axis=1, keepdims=True)
    p = jnp.exp(p)
    p = p / jnp.sum(p, axis=1, keepdims=True)
    p2 = p - jnp.max(p, axis=1, keepdims=True)
    p2 = jnp.exp(p2)
    p2 = p2 / jnp.sum(p2, axis=1, keepdims=True)

    rows = jax.lax.dot_general(
        p2.astype(jnp.bfloat16), v, (((1,), (0,)), ((), ())),
        preferred_element_type=jnp.float32)
    rows_aug = jnp.concatenate(
        [rows, jnp.ones((_U, _D), jnp.float32)], axis=1)

    local_idx = (idx_ref[0, 0:1, :] - h * _N).astype(jnp.float32)
    match = (idx0_ref[...] == local_idx).astype(jnp.float32)
    aug_ref[h] = jax.lax.dot_general(
        match, rows_aug, (((1,), (0,)), ((), ())),
        preferred_element_type=jnp.float32)

    @pl.when(h == _H - 1)
    def _():
        inter = aug_ref[0, :, _D:_D + 1]
        for g in range(1, _H):
            inter = jnp.minimum(inter, aug_ref[g, :, _D:_D + 1])
        aug2_ref[...] = jnp.concatenate(
            [jnp.zeros((_U, _C), jnp.float32),
             inter * jnp.ones((_U, _D), jnp.float32)], axis=1)

        @pl.when(jnp.max(inter) > 0.5)
        def _():
            cp = pltpu.make_async_copy(w_ref, wsc_ref, sem)
            cp.start()
            cp.wait()
            cand = jnp.concatenate(
                [aug_ref[g, :, :_D] for g in range(_H)], axis=1)
            aug2_ref[:, :_C] = jax.lax.dot_general(
                cand, wsc_ref[...], (((1,), (0,)), ((), ())),
                preferred_element_type=jnp.float32) + b_ref[...]

        aug2 = aug2_ref[...]
        idx0 = idx0h_ref[0, 0:1, :].astype(jnp.float32)
        row = jax.lax.broadcasted_iota(
            jnp.int32, (_N, _U), 0).astype(jnp.float32)
        onehot = (row == idx0).astype(jnp.float32)
        scat = jax.lax.dot_general(
            onehot, aug2, (((1,), (0,)), ((), ())),
            preferred_element_type=jnp.float32)
        out_ref[...] = jnp.where(
            scat[:, _C:_C + 1] > 0.5, scat[:, :_C], jnp.nan)


def _attn_fc(q_red, k, v, idx_flat, idx0_col, w, b):
    return pl.pallas_call(
        _attn_fc_kernel,
        grid=(_H,),
        in_specs=[
            pl.BlockSpec((1, _U, _D), lambda h: (h, 0, 0)),
            pl.BlockSpec((1, _N, _D), lambda h: (h, 0, 0)),
            pl.BlockSpec((1, _N, _D), lambda h: (h, 0, 0)),
            pl.BlockSpec((1, 1, _U), lambda h: (h, 0, 0)),
            pl.BlockSpec((1, 1, _U), lambda h: (0, 0, 0)),
            pl.BlockSpec((_U, 1), lambda h: (0, 0)),
            pl.BlockSpec(memory_space=pl.ANY),
            pl.BlockSpec((1, _C), lambda h: (0, 0)),
        ],
        out_specs=pl.BlockSpec((_N, _C), lambda h: (0, 0)),
        out_shape=jax.ShapeDtypeStruct((_N, _C), jnp.float32),
        scratch_shapes=[
            pltpu.VMEM((_H, _U, 2 * _D), jnp.float32),
            pltpu.VMEM((_C, _C), jnp.float32),
            pltpu.VMEM((_U, _C + _D), jnp.float32),
            pltpu.SemaphoreType.DMA,
        ],
    )(q_red, k, v, idx_flat, idx_flat, idx0_col, w, b)


@jax.jit
def kernel(query, W_qkv, b_qkv, W_fc, b_fc):
    B, N, C = query.shape
    x = query.reshape(N, C)
    q, k, v, idx3, idx0_col = _qkv_proj(x, W_qkv, b_qkv.reshape(12, 1, 512))
    idx_flat = idx3.reshape(_H, 1, _U)
    q_red = _sc_gather(q.reshape(_H * _N, _D), idx3.reshape(_H * _U))
    out = _attn_fc(q_red.reshape(_H, _U, _D), k, v, idx_flat, idx0_col,
                   W_fc, b_fc.reshape(1, C))
    return out.reshape(B, N, C)


# speedup vs baseline: 1.1074x; 1.0195x over previous
import functools

import jax
import jax.numpy as jnp
from jax import lax
from jax.experimental import pallas as pl
from jax.experimental.pallas import tpu as pltpu
from jax.experimental.pallas import tpu_sc as plsc

_N = 2048
_C = 2048
_H = 16
_D = 128
_U = 40
_SCALE = _D ** -0.5


def _qkv_kernel(x_ref, w_ref, b_ref, q_ref, k_ref, v_ref, idx_ref, idx0_ref,
                xbf_ref, n2_ref):
    hh = pl.program_id(0)
    s = pl.program_id(1)

    @pl.when((hh == 0) & (s == 0))
    def _():
        xbf_ref[...] = x_ref[...].astype(jnp.bfloat16)

    acc = jax.lax.dot_general(
        xbf_ref[...], w_ref[...].astype(jnp.bfloat16), (((1,), (0,)), ((), ())),
        preferred_element_type=jnp.float32)
    acc = acc + b_ref[0]

    @pl.when(s == 0)
    def _():
        for j in range(4):
            q_ref[j, :, :] = acc[:, j * _D:(j + 1) * _D]
        sq = acc * acc
        cj = jax.lax.broadcasted_iota(jnp.int32, (4, 512), 0)
        cc = jax.lax.broadcasted_iota(jnp.int32, (4, 512), 1)
        sel = (cc // _D == cj).astype(jnp.float32)
        n2_ref[hh] = jax.lax.dot_general(
            sel, sq, (((1,), (1,)), ((), ())), preferred_element_type=jnp.float32)

    @pl.when(s == 1)
    def _():
        for j in range(4):
            k_ref[j, :, :] = acc[:, j * _D:(j + 1) * _D].astype(jnp.bfloat16)

    @pl.when(s == 2)
    def _():
        for j in range(4):
            v_ref[j, :, :] = acc[:, j * _D:(j + 1) * _D].astype(jnp.bfloat16)

    @pl.when((hh == 3) & (s == 2))
    def _():
        bits = jax.lax.bitcast_convert_type(n2_ref[...], jnp.int32)
        col = jax.lax.broadcasted_iota(jnp.int32, (4, 4, _N), 2)
        keys = (bits & ~jnp.int32(2047)) | (jnp.int32(2047) - col)
        picks = []
        for _ in range(_U):
            m = jnp.max(keys, axis=2, keepdims=True)
            picks.append(m)
            keys = jnp.where(keys == m, jnp.iinfo(jnp.int32).min, keys)
        mkeys = jnp.concatenate(picks, axis=2)
        idx = jnp.int32(2047) - (mkeys & jnp.int32(2047))
        head = (jax.lax.broadcasted_iota(jnp.int32, (4, 4, _U), 0) * 4
                + jax.lax.broadcasted_iota(jnp.int32, (4, 4, _U), 1))
        idx_ref[...] = idx + head * _N
        one = jnp.ones((1, 1), jnp.float32)
        idx0_ref[...] = jax.lax.dot_general(
            idx[0, 0:1, :].astype(jnp.float32), one,
            (((0,), (0,)), ((), ())), preferred_element_type=jnp.float32)


def _qkv_proj(x, w, b):
    return pl.pallas_call(
        _qkv_kernel,
        grid=(4, 3),
        in_specs=[
            pl.BlockSpec((_N, _C), lambda hh, s: (0, 0)),
            pl.BlockSpec((_C, 512), lambda hh, s: (0, s * 4 + hh)),
            pl.BlockSpec((1, 1, 512), lambda hh, s: (s * 4 + hh, 0, 0)),
        ],
        out_specs=[
            pl.BlockSpec((4, _N, _D), lambda hh, s: (hh, 0, 0)),
            pl.BlockSpec((4, _N, _D), lambda hh, s: (hh, 0, 0)),
            pl.BlockSpec((4, _N, _D), lambda hh, s: (hh, 0, 0)),
            pl.BlockSpec((4, 4, _U), lambda hh, s: (0, 0, 0)),
            pl.BlockSpec((_U, 1), lambda hh, s: (0, 0)),
        ],
        out_shape=[
            jax.ShapeDtypeStruct((_H, _N, _D), jnp.float32),
            jax.ShapeDtypeStruct((_H, _N, _D), jnp.bfloat16),
            jax.ShapeDtypeStruct((_H, _N, _D), jnp.bfloat16),
            jax.ShapeDtypeStruct((4, 4, _U), jnp.int32),
            jax.ShapeDtypeStruct((_U, 1), jnp.float32),
        ],
        scratch_shapes=[
            pltpu.VMEM((_N, _C), jnp.bfloat16),
            pltpu.VMEM((4, 4, _N), jnp.float32),
        ],
    )(x, w, b)


def _sc_gather(q_rows, idx_flat):
    mesh = plsc.VectorSubcoreMesh(core_axis_name="c", subcore_axis_name="s")

    @functools.partial(
        pl.kernel, mesh=mesh,
        out_type=jax.ShapeDtypeStruct((_H * _U, _D), jnp.float32),
        scratch_types=[
            pltpu.VMEM((_U,), jnp.int32),
            pltpu.VMEM((_U, _D), jnp.float32),
            pltpu.SemaphoreType.DMA,
        ],
    )
    def gather(table_hbm, idx_hbm, out_hbm, idx_v, rows_v, sem):
        wid = lax.axis_index("s") * 2 + lax.axis_index("c")

        @pl.when(wid < _H)
        def _():
            base = wid * _U
            pltpu.sync_copy(idx_hbm.at[pl.ds(base, _U)], idx_v)
            pltpu.async_copy(table_hbm.at[idx_v], rows_v, sem).wait()
            pltpu.sync_copy(rows_v, out_hbm.at[pl.ds(base, _U)])

    return gather(q_rows, idx_flat)


def _attn_fc_kernel(qred_ref, k_ref, v_ref, idx_ref, idx0h_ref, idx0_ref,
                    w_ref, b_ref, out_ref, aug_ref, wsc_ref, aug2_ref, sem):
    h = pl.program_id(0)
    q_red = qred_ref[0].astype(jnp.bfloat16)
    k = k_ref[0]
    v = v_ref[0]
    s = jax.lax.dot_general(
        q_red, k, (((1,), (1,)), ((), ())),
        preferred_element_type=jnp.float32) * _SCALE

    p = s - jnp.max(s, axis=1, keepdims=True)
    p = jnp.exp(p)
    p = p / jnp.sum(p, axis=1, keepdims=True)
    p2 = p - jnp.max(p, axis=1, keepdims=True)
    p2 = jnp.exp(p2)
    p2 = p2 / jnp.sum(p2, axis=1, keepdims=True)

    rows = jax.lax.dot_general(
        p2.astype(jnp.bfloat16), v, (((1,), (0,)), ((), ())),
        preferred_element_type=jnp.float32)
    rows_aug = jnp.concatenate(
        [rows, jnp.ones((_U, _D), jnp.float32)], axis=1)

    local_idx = (idx_ref[0, 0:1, :] - h * _N).astype(jnp.float32)
    match = (idx0_ref[...] == local_idx).astype(jnp.float32)
    aug_ref[h] = jax.lax.dot_general(
        match, rows_aug, (((1,), (0,)), ((), ())),
        preferred_element_type=jnp.float32)

    @pl.when(h == _H - 1)
    def _():
        inter = aug_ref[0, :, _D:_D + 1]
        for g in range(1, _H):
            inter = jnp.minimum(inter, aug_ref[g, :, _D:_D + 1])
        aug2_ref[...] = jnp.concatenate(
            [jnp.zeros((_U, _C), jnp.float32),
             inter * jnp.ones((_U, _D), jnp.float32)], axis=1)

        @pl.when(jnp.max(inter) > 0.5)
        def _():
            cp = pltpu.make_async_copy(w_ref, wsc_ref, sem)
            cp.start()
            cp.wait()
            cand = jnp.concatenate(
                [aug_ref[g, :, :_D] for g in range(_H)], axis=1)
            aug2_ref[:, :_C] = jax.lax.dot_general(
                cand, wsc_ref[...], (((1,), (0,)), ((), ())),
                preferred_element_type=jnp.float32) + b_ref[...]

        aug2 = aug2_ref[...]
        idx0 = idx0h_ref[0, 0:1, :].astype(jnp.float32)
        row = jax.lax.broadcasted_iota(
            jnp.int32, (_N, _U), 0).astype(jnp.float32)
        onehot = (row == idx0).astype(jnp.float32)
        scat = jax.lax.dot_general(
            onehot, aug2, (((1,), (0,)), ((), ())),
            preferred_element_type=jnp.float32)
        out_ref[...] = jnp.where(
            scat[:, _C:_C + 1] > 0.5, scat[:, :_C], jnp.nan)


def _attn_fc(q_red, k, v, idx_flat, idx0_col, w, b):
    return pl.pallas_call(
        _attn_fc_kernel,
        grid=(_H,),
        in_specs=[
            pl.BlockSpec((1, _U, _D), lambda h: (h, 0, 0)),
            pl.BlockSpec((1, _N, _D), lambda h: (h, 0, 0)),
            pl.BlockSpec((1, _N, _D), lambda h: (h, 0, 0)),
            pl.BlockSpec((1, 1, _U), lambda h: (h, 0, 0)),
            pl.BlockSpec((1, 1, _U), lambda h: (0, 0, 0)),
            pl.BlockSpec((_U, 1), lambda h: (0, 0)),
            pl.BlockSpec(memory_space=pl.ANY),
            pl.BlockSpec((1, _C), lambda h: (0, 0)),
        ],
        out_specs=pl.BlockSpec((_N, _C), lambda h: (0, 0)),
        out_shape=jax.ShapeDtypeStruct((_N, _C), jnp.float32),
        scratch_shapes=[
            pltpu.VMEM((_H, _U, 2 * _D), jnp.float32),
            pltpu.VMEM((_C, _C), jnp.float32),
            pltpu.VMEM((_U, _C + _D), jnp.float32),
            pltpu.SemaphoreType.DMA,
        ],
    )(q_red, k, v, idx_flat, idx_flat, idx0_col, w, b)


@jax.jit
def kernel(query, W_qkv, b_qkv, W_fc, b_fc):
    B, N, C = query.shape
    x = query.reshape(N, C)
    q, k, v, idx3, idx0_col = _qkv_proj(x, W_qkv, b_qkv.reshape(12, 1, 512))
    idx_flat = idx3.reshape(_H, 1, _U)
    q_red = _sc_gather(q.reshape(_H * _N, _D), idx3.reshape(_H * _U))
    out = _attn_fc(q_red.reshape(_H, _U, _D), k, v, idx_flat, idx0_col,
                   W_fc, b_fc.reshape(1, C))
    return out.reshape(B, N, C)


# speedup vs baseline: 1.1261x; 1.0169x over previous
import functools

import jax
import jax.numpy as jnp
from jax import lax
from jax.experimental import pallas as pl
from jax.experimental.pallas import tpu as pltpu
from jax.experimental.pallas import tpu_sc as plsc

_N = 2048
_C = 2048
_H = 16
_D = 128
_U = 40
_SCALE = _D ** -0.5


def _qkv_kernel(x_ref, w_ref, b_ref, q_ref, k_ref, v_ref, idx_ref, idx0_ref,
                xbf_ref, n2_ref):
    hh = pl.program_id(0)
    s = pl.program_id(1)

    @pl.when((hh == 0) & (s == 0))
    def _():
        xbf_ref[...] = x_ref[...].astype(jnp.bfloat16)

    acc = jax.lax.dot_general(
        xbf_ref[...], w_ref[...].astype(jnp.bfloat16), (((1,), (0,)), ((), ())),
        preferred_element_type=jnp.float32)
    acc = acc + b_ref[0]

    @pl.when(s == 0)
    def _():
        for j in range(4):
            q_ref[j, :, :] = acc[:, j * _D:(j + 1) * _D]
        sq = acc * acc
        cj = jax.lax.broadcasted_iota(jnp.int32, (4, 512), 0)
        cc = jax.lax.broadcasted_iota(jnp.int32, (4, 512), 1)
        sel = (cc // _D == cj).astype(jnp.float32)
        n2_ref[hh] = jax.lax.dot_general(
            sel, sq, (((1,), (1,)), ((), ())), preferred_element_type=jnp.float32)

    @pl.when(s == 1)
    def _():
        for j in range(4):
            k_ref[j, :, :] = acc[:, j * _D:(j + 1) * _D].astype(jnp.bfloat16)

    @pl.when(s == 2)
    def _():
        for j in range(4):
            v_ref[j, :, :] = acc[:, j * _D:(j + 1) * _D].astype(jnp.bfloat16)

    @pl.when((hh == 3) & (s == 2))
    def _():
        bits = jax.lax.bitcast_convert_type(n2_ref[...], jnp.int32)
        col = jax.lax.broadcasted_iota(jnp.int32, (4, 4, _N), 2)
        keys = (bits & ~jnp.int32(2047)) | (jnp.int32(2047) - col)
        picks = []
        for _ in range(_U):
            m = jnp.max(keys, axis=2, keepdims=True)
            picks.append(m)
            keys = jnp.where(keys == m, jnp.iinfo(jnp.int32).min, keys)
        mkeys = jnp.concatenate(picks, axis=2)
        idx = jnp.int32(2047) - (mkeys & jnp.int32(2047))
        head = (jax.lax.broadcasted_iota(jnp.int32, (4, 4, _U), 0) * 4
                + jax.lax.broadcasted_iota(jnp.int32, (4, 4, _U), 1))
        idx_ref[...] = idx + head * _N
        one = jnp.ones((1, 1), jnp.float32)
        idx0_ref[...] = jax.lax.dot_general(
            idx[0, 0:1, :].astype(jnp.float32), one,
            (((0,), (0,)), ((), ())), preferred_element_type=jnp.float32)


def _qkv_proj(x, w, b):
    return pl.pallas_call(
        _qkv_kernel,
        grid=(4, 3),
        in_specs=[
            pl.BlockSpec((_N, _C), lambda hh, s: (0, 0)),
            pl.BlockSpec((_C, 512), lambda hh, s: (0, s * 4 + hh)),
            pl.BlockSpec((1, 1, 512), lambda hh, s: (s * 4 + hh, 0, 0)),
        ],
        out_specs=[
            pl.BlockSpec((4, _N, _D), lambda hh, s: (hh, 0, 0)),
            pl.BlockSpec((4, _N, _D), lambda hh, s: (hh, 0, 0)),
            pl.BlockSpec((4, _N, _D), lambda hh, s: (hh, 0, 0)),
            pl.BlockSpec((4, 4, _U), lambda hh, s: (0, 0, 0)),
            pl.BlockSpec((_U, 1), lambda hh, s: (0, 0)),
        ],
        out_shape=[
            jax.ShapeDtypeStruct((_H, _N, _D), jnp.float32),
            jax.ShapeDtypeStruct((_H, _N, _D), jnp.bfloat16),
            jax.ShapeDtypeStruct((_H, _N, _D), jnp.bfloat16),
            jax.ShapeDtypeStruct((4, 4, _U), jnp.int32),
            jax.ShapeDtypeStruct((_U, 1), jnp.float32),
        ],
        scratch_shapes=[
            pltpu.VMEM((_N, _C), jnp.bfloat16),
            pltpu.VMEM((4, 4, _N), jnp.float32),
        ],
    )(x, w, b)


def _sc_gather(q_rows, idx_flat):
    mesh = plsc.VectorSubcoreMesh(core_axis_name="c", subcore_axis_name="s")

    @functools.partial(
        pl.kernel, mesh=mesh,
        out_type=jax.ShapeDtypeStruct((_H * _U, _D), jnp.float32),
        scratch_types=[
            pltpu.VMEM((_U,), jnp.int32),
            pltpu.VMEM((_U, _D), jnp.float32),
            pltpu.SemaphoreType.DMA,
        ],
    )
    def gather(table_hbm, idx_hbm, out_hbm, idx_v, rows_v, sem):
        wid = lax.axis_index("s") * 2 + lax.axis_index("c")

        @pl.when(wid < _H)
        def _():
            base = wid * _U
            pltpu.sync_copy(idx_hbm.at[pl.ds(base, _U)], idx_v)
            pltpu.async_copy(table_hbm.at[idx_v], rows_v, sem).wait()
            pltpu.sync_copy(rows_v, out_hbm.at[pl.ds(base, _U)])

    return gather(q_rows, idx_flat)


def _attn_fc_kernel(qred_ref, k_ref, v_ref, idx_ref, idx0h_ref, idx0_ref,
                    w_ref, b_ref, out_ref, aug_ref, wsc_ref, aug2_ref, sem):
    g = pl.program_id(0)
    for j in range(2):
        h = g * 2 + j
        q_red = qred_ref[j].astype(jnp.bfloat16)
        k = k_ref[j]
        v = v_ref[j]
        s = jax.lax.dot_general(
            q_red, k, (((1,), (1,)), ((), ())),
            preferred_element_type=jnp.float32) * _SCALE

        p = s - jnp.max(s, axis=1, keepdims=True)
        p = jnp.exp(p)
        p = p / jnp.sum(p, axis=1, keepdims=True)
        p2 = p - jnp.max(p, axis=1, keepdims=True)
        p2 = jnp.exp(p2)
        p2 = p2 / jnp.sum(p2, axis=1, keepdims=True)

        rows = jax.lax.dot_general(
            p2.astype(jnp.bfloat16), v, (((1,), (0,)), ((), ())),
            preferred_element_type=jnp.float32)
        rows_aug = jnp.concatenate(
            [rows, jnp.ones((_U, _D), jnp.float32)], axis=1)

        local_idx = (idx_ref[j, 0:1, :] - h * _N).astype(jnp.float32)
        match = (idx0_ref[...] == local_idx).astype(jnp.float32)
        aug_ref[h] = jax.lax.dot_general(
            match, rows_aug, (((1,), (0,)), ((), ())),
            preferred_element_type=jnp.float32)

    @pl.when(g == _H // 2 - 1)
    def _():
        inter = aug_ref[0, :, _D:_D + 1]
        for g in range(1, _H):
            inter = jnp.minimum(inter, aug_ref[g, :, _D:_D + 1])
        aug2_ref[...] = jnp.concatenate(
            [jnp.zeros((_U, _C), jnp.float32),
             inter * jnp.ones((_U, _D), jnp.float32)], axis=1)

        @pl.when(jnp.max(inter) > 0.5)
        def _():
            cp = pltpu.make_async_copy(w_ref, wsc_ref, sem)
            cp.start()
            cp.wait()
            cand = jnp.concatenate(
                [aug_ref[g, :, :_D] for g in range(_H)], axis=1)
            aug2_ref[:, :_C] = jax.lax.dot_general(
                cand, wsc_ref[...], (((1,), (0,)), ((), ())),
                preferred_element_type=jnp.float32) + b_ref[...]

        aug2 = aug2_ref[...]
        idx0 = idx0h_ref[0, 0:1, :].astype(jnp.float32)
        row = jax.lax.broadcasted_iota(
            jnp.int32, (_N, _U), 0).astype(jnp.float32)
        onehot = (row == idx0).astype(jnp.float32)
        scat = jax.lax.dot_general(
            onehot, aug2, (((1,), (0,)), ((), ())),
            preferred_element_type=jnp.float32)
        out_ref[...] = jnp.where(
            scat[:, _C:_C + 1] > 0.5, scat[:, :_C], jnp.nan)


def _attn_fc(q_red, k, v, idx_flat, idx0_col, w, b):
    return pl.pallas_call(
        _attn_fc_kernel,
        grid=(_H // 2,),
        in_specs=[
            pl.BlockSpec((2, _U, _D), lambda g: (g, 0, 0)),
            pl.BlockSpec((2, _N, _D), lambda g: (g, 0, 0)),
            pl.BlockSpec((2, _N, _D), lambda g: (g, 0, 0)),
            pl.BlockSpec((2, 1, _U), lambda g: (g, 0, 0)),
            pl.BlockSpec((1, 1, _U), lambda g: (0, 0, 0)),
            pl.BlockSpec((_U, 1), lambda g: (0, 0)),
            pl.BlockSpec(memory_space=pl.ANY),
            pl.BlockSpec((1, _C), lambda g: (0, 0)),
        ],
        out_specs=pl.BlockSpec((_N, _C), lambda g: (0, 0)),
        out_shape=jax.ShapeDtypeStruct((_N, _C), jnp.float32),
        scratch_shapes=[
            pltpu.VMEM((_H, _U, 2 * _D), jnp.float32),
            pltpu.VMEM((_C, _C), jnp.float32),
            pltpu.VMEM((_U, _C + _D), jnp.float32),
            pltpu.SemaphoreType.DMA,
        ],
    )(q_red, k, v, idx_flat, idx_flat, idx0_col, w, b)


@jax.jit
def kernel(query, W_qkv, b_qkv, W_fc, b_fc):
    B, N, C = query.shape
    x = query.reshape(N, C)
    q, k, v, idx3, idx0_col = _qkv_proj(x, W_qkv, b_qkv.reshape(12, 1, 512))
    idx_flat = idx3.reshape(_H, 1, _U)
    q_red = _sc_gather(q.reshape(_H * _N, _D), idx3.reshape(_H * _U))
    out = _attn_fc(q_red.reshape(_H, _U, _D), k, v, idx_flat, idx0_col,
                   W_fc, b_fc.reshape(1, C))
    return out.reshape(B, N, C)
